# baseline (speedup 1.0000x reference)
import functools

import jax
import jax.numpy as jnp
from jax import lax
from jax.experimental import pallas as pl
from jax.experimental.pallas import tpu as pltpu
from jax.experimental.pallas import tpu_sc as plsc

_BATCH = 16384
_EMB = 16
_OUT = 10
_NIG = 10
_NGG = 21
_TROWS = 12500

_info = plsc.get_sparse_core_info()
_NC = _info.num_cores
_NS = _info.num_subcores
_NW = _NC * _NS
_BPW = _BATCH // _NW
_OPW = _BPW // 8

_sc_mesh = plsc.VectorSubcoreMesh(core_axis_name="c", subcore_axis_name="s")


@functools.partial(
    pl.kernel,
    mesh=_sc_mesh,
    out_type=jax.ShapeDtypeStruct((_BATCH // 8, 128), jnp.float32),
    scratch_types=[
        pltpu.VMEM((_BPW,), jnp.int32),
        pltpu.VMEM((_BPW,), jnp.int32),
        pltpu.VMEM((_BPW, 128), jnp.float32),
        pltpu.VMEM((_OPW, 128), jnp.float32),
        pltpu.SemaphoreType.DMA,
    ],
    compiler_params=pltpu.CompilerParams(needs_layout_passes=False),
)
def _sc_gather(table_hbm, idx_hbm, out_hbm, item_v, row_v, raw_v, out_v, sem):
    wid = lax.axis_index("s") * _NC + lax.axis_index("c")
    base = wid * _BPW
    pltpu.sync_copy(idx_hbm.at[pl.ds(base, _BPW)], item_v)

    lanes = lax.iota(jnp.int32, 16)

    def _rows(i, _):
        item16 = item_v[pl.ds(i * 16, 16)]
        row_v[pl.ds(i * 16, 16)] = lax.shift_right_logical(item16, 3)
        return ()

    lax.fori_loop(0, _BPW // 16, _rows, (), unroll=4)

    pltpu.async_copy(table_hbm.at[row_v], raw_v, sem).wait()

    def _extract(r, _):
        splat = jnp.full((16,), r, jnp.int32)
        item_r = plsc.load_gather(item_v, [splat])
        col = (item_r & 7) * 16 + lanes
        val = plsc.load_gather(raw_v, [splat, col])
        orow = jnp.full((16,), lax.shift_right_logical(r, 3), jnp.int32)
        ocol = (r & 7) * 16 + lanes
        plsc.store_scatter(out_v, [orow, ocol], val)
        return ()

    lax.fori_loop(0, _BPW, _extract, (), unroll=8)

    pltpu.sync_copy(out_v, out_hbm.at[pl.ds(wid * _OPW, _OPW)])


_ROWS = 2048


def _tc_body(g_ref, ig_ref, gg_ref, w0_ref, wig_ref, wgg_ref, b_ref, o_ref):
    g = jnp.maximum(g_ref[...], 0.0)
    acc = jnp.dot(g, w0_ref[...], preferred_element_type=jnp.float32)
    oh_ig = (lax.broadcasted_iota(jnp.int32, (_ROWS, _NIG), 1)
             == ig_ref[...]).astype(jnp.float32)
    acc += jnp.dot(oh_ig, wig_ref[...], preferred_element_type=jnp.float32)
    oh_gg = (lax.broadcasted_iota(jnp.int32, (_ROWS, _NGG), 1)
             == gg_ref[...]).astype(jnp.float32)
    acc += jnp.dot(oh_gg, wgg_ref[...], preferred_element_type=jnp.float32)
    o_ref[...] = acc + b_ref[...]


def _tc_dense(g, ig2, gg2, w0, wig, wgg, b2):
    grid = _BATCH // _ROWS
    return pl.pallas_call(
        _tc_body,
        grid=(grid,),
        in_specs=[
            pl.BlockSpec((_ROWS, _EMB), lambda i: (i, 0)),
            pl.BlockSpec((_ROWS, 1), lambda i: (i, 0)),
            pl.BlockSpec((_ROWS, 1), lambda i: (i, 0)),
            pl.BlockSpec((_EMB, _OUT), lambda i: (0, 0)),
            pl.BlockSpec((_NIG, _OUT), lambda i: (0, 0)),
            pl.BlockSpec((_NGG, _OUT), lambda i: (0, 0)),
            pl.BlockSpec((1, _OUT), lambda i: (0, 0)),
        ],
        out_specs=pl.BlockSpec((_ROWS, _OUT), lambda i: (i, 0)),
        out_shape=jax.ShapeDtypeStruct((_BATCH, _OUT), jnp.float32),
    )(g, ig2, gg2, w0, wig, wgg, b2)


@jax.jit
def kernel(item_indices, index_group_indices, garment_group_indices, emb_table, W, b):
    item = item_indices.astype(jnp.int32)
    ig2 = index_group_indices.astype(jnp.int32).reshape(_BATCH, 1)
    gg2 = garment_group_indices.astype(jnp.int32).reshape(_BATCH, 1)
    t128 = emb_table.reshape(_TROWS, 128)
    g = _sc_gather(t128, item).reshape(_BATCH, _EMB)
    w0 = W[:_EMB]
    wig = W[_EMB:_EMB + _NIG]
    wgg = W[_EMB + _NIG:]
    return _tc_dense(g, ig2, gg2, w0, wig, wgg, b.reshape(1, _OUT))


# speedup vs baseline: 1.3064x; 1.3064x over previous
import functools

import jax
import jax.numpy as jnp
from jax import lax
from jax.experimental import pallas as pl
from jax.experimental.pallas import tpu as pltpu
from jax.experimental.pallas import tpu_sc as plsc

_BATCH = 16384
_EMB = 16
_OUT = 10
_NIG = 10
_NGG = 21

_info = plsc.get_sparse_core_info()
_NC = _info.num_cores
_NS = _info.num_subcores
_NW = _NC * _NS
_BPW = _BATCH // _NW
_NBLK = _BPW // 16

_W0 = 0
_WIG = 256
_WGG = 512

_sc_mesh = plsc.VectorSubcoreMesh(core_axis_name="c", subcore_axis_name="s")


@functools.partial(
    pl.kernel,
    mesh=_sc_mesh,
    out_type=jax.ShapeDtypeStruct((_OUT, _BATCH), jnp.float32),
    scratch_types=[
        pltpu.VMEM((_BPW,), jnp.int32),
        pltpu.VMEM((_BPW,), jnp.int32),
        pltpu.VMEM((_BPW,), jnp.int32),
        pltpu.VMEM((1024,), jnp.float32),
        pltpu.VMEM((_EMB * _OUT, 16), jnp.float32),
        pltpu.VMEM((_BPW, _EMB), jnp.float32),
        pltpu.VMEM((_OUT, _BPW), jnp.float32),
        pltpu.SemaphoreType.DMA,
    ],
    compiler_params=pltpu.CompilerParams(
        use_tc_tiling_on_sc=False, needs_layout_passes=False),
)
def _sc_tower(table_hbm, item_hbm, ig_hbm, gg_hbm, params_hbm, out_hbm,
              item_v, ig_v, gg_v, params_v, wsplat_v, raw_v, out_v, sem):
    wid = lax.axis_index("s") * _NC + lax.axis_index("c")
    base = wid * _BPW
    pltpu.sync_copy(item_hbm.at[pl.ds(base, _BPW)], item_v)
    pltpu.sync_copy(ig_hbm.at[pl.ds(base, _BPW)], ig_v)
    pltpu.sync_copy(gg_hbm.at[pl.ds(base, _BPW)], gg_v)
    pltpu.sync_copy(params_hbm, params_v)

    gather = pltpu.async_copy(table_hbm.at[item_v], raw_v, sem)

    def _splat(ck, _):
        wsplat_v[ck, :] = plsc.load_gather(params_v, [jnp.full((16,), ck, jnp.int32)])
        return ()

    lax.fori_loop(0, _EMB * _OUT, _splat, (), unroll=4)

    gather.wait()

    lanes = lax.iota(jnp.int32, 16)

    def _block(b, _):
        rows16 = b * 16 + lanes
        cols = [
            jnp.maximum(
                plsc.load_gather(raw_v, [rows16, jnp.full((16,), k, jnp.int32)]),
                0.0)
            for k in range(_EMB)
        ]
        ig16 = ig_v[pl.ds(b * 16, 16)]
        gg16 = gg_v[pl.ds(b * 16, 16)]
        for c in range(_OUT):
            acc = plsc.load_gather(params_v, [_WIG + c * _NIG + ig16])
            acc = acc + plsc.load_gather(params_v, [_WGG + c * _NGG + gg16])
            for k in range(_EMB):
                acc = acc + cols[k] * wsplat_v[c * _EMB + k, :]
            out_v[c, pl.ds(b * 16, 16)] = acc
        return ()

    lax.fori_loop(0, _NBLK, _block, ())

    pltpu.sync_copy(out_v, out_hbm.at[:, pl.ds(base, _BPW)])


@jax.jit
def kernel(item_indices, index_group_indices, garment_group_indices, emb_table, W, b):
    item = item_indices.astype(jnp.int32)
    ig = index_group_indices.astype(jnp.int32)
    gg = garment_group_indices.astype(jnp.int32)
    w0t = W[:_EMB].T.reshape(-1)
    wigb = (W[_EMB:_EMB + _NIG] + b[None, :]).T.reshape(-1)
    wggt = W[_EMB + _NIG:].T.reshape(-1)
    params = jnp.zeros((1024,), jnp.float32)
    params = lax.dynamic_update_slice(params, w0t, (_W0,))
    params = lax.dynamic_update_slice(params, wigb, (_WIG,))
    params = lax.dynamic_update_slice(params, wggt, (_WGG,))
    out_t = _sc_tower(emb_table, item, ig, gg, params)
    return out_t.T


# speedup vs baseline: 1.3071x; 1.0006x over previous
import functools

import jax
import jax.numpy as jnp
from jax import lax
from jax.experimental import pallas as pl
from jax.experimental.pallas import tpu as pltpu
from jax.experimental.pallas import tpu_sc as plsc

_BATCH = 16384
_EMB = 16
_OUT = 10
_NIG = 10
_NGG = 21
_TBLK = 2048
_TGRID = 49
_TROWS = _TGRID * _TBLK // 8

_info = plsc.get_sparse_core_info()
_NC = _info.num_cores
_NS = _info.num_subcores
_NW = _NC * _NS
_BPW = _BATCH // _NW
_NBLK = _BPW // 16

_W0 = 0
_WIG = 256
_WGG = 512


def _repack_body(t_ref, o_ref):
    x = t_ref[...]
    for s in range(8):
        o_ref[:, s * _EMB:(s + 1) * _EMB] = jnp.transpose(
            x[:, s * 256:(s + 1) * 256])


def _repack(tT):
    return pl.pallas_call(
        _repack_body,
        grid=(_TGRID,),
        in_specs=[pl.BlockSpec((_EMB, _TBLK), lambda i: (0, i))],
        out_specs=pl.BlockSpec((_TBLK // 8, 128), lambda i: (i, 0)),
        out_shape=jax.ShapeDtypeStruct((_TROWS, 128), jnp.float32),
    )(tT)


_sc_mesh = plsc.VectorSubcoreMesh(core_axis_name="c", subcore_axis_name="s")


@functools.partial(
    pl.kernel,
    mesh=_sc_mesh,
    out_type=jax.ShapeDtypeStruct((_OUT, _BATCH), jnp.float32),
    scratch_types=[
        pltpu.VMEM((_BPW,), jnp.int32),
        pltpu.VMEM((_BPW,), jnp.int32),
        pltpu.VMEM((_BPW,), jnp.int32),
        pltpu.VMEM((_BPW,), jnp.int32),
        pltpu.VMEM((1024,), jnp.float32),
        pltpu.VMEM((_EMB * _OUT, 16), jnp.float32),
        pltpu.VMEM((_BPW, 128), jnp.float32),
        pltpu.VMEM((_OUT, _BPW), jnp.float32),
        pltpu.SemaphoreType.DMA,
    ],
    compiler_params=pltpu.CompilerParams(
        use_tc_tiling_on_sc=False, needs_layout_passes=False),
)
def _sc_tower(table_hbm, item_hbm, ig_hbm, gg_hbm, params_hbm, out_hbm,
              item_v, row_v, ig_v, gg_v, params_v, wsplat_v, raw_v, out_v,
              sem):
    wid = lax.axis_index("s") * _NC + lax.axis_index("c")
    base = wid * _BPW
    pltpu.sync_copy(item_hbm.at[pl.ds(base, _BPW)], item_v)
    pltpu.sync_copy(ig_hbm.at[pl.ds(base, _BPW)], ig_v)
    pltpu.sync_copy(gg_hbm.at[pl.ds(base, _BPW)], gg_v)
    pltpu.sync_copy(params_hbm, params_v)

    def _rows(i, _):
        item16 = item_v[pl.ds(i * 16, 16)]
        row_v[pl.ds(i * 16, 16)] = (
            lax.shift_left(lax.shift_right_logical(item16, 11), 8)
            + (item16 & 255))
        return ()

    lax.fori_loop(0, _BPW // 16, _rows, (), unroll=4)

    gather = pltpu.async_copy(table_hbm.at[row_v], raw_v, sem)

    def _splat(ck, _):
        wsplat_v[ck, :] = plsc.load_gather(params_v, [jnp.full((16,), ck, jnp.int32)])
        return ()

    lax.fori_loop(0, _EMB * _OUT, _splat, (), unroll=4)

    gather.wait()

    lanes = lax.iota(jnp.int32, 16)

    def _block(b, _):
        rows16 = b * 16 + lanes
        item16 = item_v[pl.ds(b * 16, 16)]
        lane0 = (lax.shift_right_logical(item16, 8) & 7) * 16
        cols = [
            jnp.maximum(
                plsc.load_gather(raw_v, [rows16, lane0 + k]), 0.0)
            for k in range(_EMB)
        ]
        ig16 = ig_v[pl.ds(b * 16, 16)]
        gg16 = gg_v[pl.ds(b * 16, 16)]
        for c in range(_OUT):
            acc = plsc.load_gather(params_v, [_WIG + c * _NIG + ig16])
            acc = acc + plsc.load_gather(params_v, [_WGG + c * _NGG + gg16])
            for k in range(_EMB):
                acc = acc + cols[k] * wsplat_v[c * _EMB + k, :]
            out_v[c, pl.ds(b * 16, 16)] = acc
        return ()

    lax.fori_loop(0, _NBLK, _block, ())

    pltpu.sync_copy(out_v, out_hbm.at[:, pl.ds(base, _BPW)])


@jax.jit
def kernel(item_indices, index_group_indices, garment_group_indices, emb_table, W, b):
    item = item_indices.astype(jnp.int32)
    ig = index_group_indices.astype(jnp.int32)
    gg = garment_group_indices.astype(jnp.int32)
    t128 = _repack(emb_table.T)
    w0t = W[:_EMB].T.reshape(-1)
    wigb = (W[_EMB:_EMB + _NIG] + b[None, :]).T.reshape(-1)
    wggt = W[_EMB + _NIG:].T.reshape(-1)
    params = jnp.zeros((1024,), jnp.float32)
    params = lax.dynamic_update_slice(params, w0t, (_W0,))
    params = lax.dynamic_update_slice(params, wigb, (_WIG,))
    params = lax.dynamic_update_slice(params, wggt, (_WGG,))
    out_t = _sc_tower(t128, item, ig, gg, params)
    return out_t.T


# speedup vs baseline: 1.5768x; 1.2063x over previous
import functools

import jax
import jax.numpy as jnp
from jax import lax
from jax.experimental import pallas as pl
from jax.experimental.pallas import tpu as pltpu
from jax.experimental.pallas import tpu_sc as plsc

_BATCH = 16384
_EMB = 16
_OUT = 10
_NIG = 10
_NGG = 21
_TBLK = 8192
_TGRID = 13
_TROWS = _TGRID * _TBLK // 8
_SLOT = _TBLK // 8

_info = plsc.get_sparse_core_info()
_NC = _info.num_cores
_NS = _info.num_subcores
_NW = _NC * _NS
_BPW = _BATCH // _NW
_NBLK = _BPW // 16

_W0 = 0
_WIG = 256
_WGG = 512


def _repack_body(t_ref, o_ref):
    y = jnp.transpose(t_ref[...])
    for s in range(8):
        o_ref[:, s * _EMB:(s + 1) * _EMB] = y[s * _SLOT:(s + 1) * _SLOT, :]


def _repack(tT):
    return pl.pallas_call(
        _repack_body,
        grid=(_TGRID,),
        in_specs=[pl.BlockSpec((_EMB, _TBLK), lambda i: (0, i))],
        out_specs=pl.BlockSpec((_TBLK // 8, 128), lambda i: (i, 0)),
        out_shape=jax.ShapeDtypeStruct((_TROWS, 128), jnp.float32),
    )(tT)


_sc_mesh = plsc.VectorSubcoreMesh(core_axis_name="c", subcore_axis_name="s")


@functools.partial(
    pl.kernel,
    mesh=_sc_mesh,
    out_type=jax.ShapeDtypeStruct((_OUT, _BATCH), jnp.float32),
    scratch_types=[
        pltpu.VMEM((_BPW,), jnp.int32),
        pltpu.VMEM((_BPW,), jnp.int32),
        pltpu.VMEM((_BPW,), jnp.int32),
        pltpu.VMEM((_BPW,), jnp.int32),
        pltpu.VMEM((1024,), jnp.float32),
        pltpu.VMEM((_EMB * _OUT, 16), jnp.float32),
        pltpu.VMEM((_BPW, _EMB), jnp.float32),
        pltpu.VMEM((_OUT, _BPW), jnp.float32),
        pltpu.SemaphoreType.DMA,
    ],
    compiler_params=pltpu.CompilerParams(
        use_tc_tiling_on_sc=False, needs_layout_passes=False),
)
def _sc_tower(table_hbm, item_hbm, ig_hbm, gg_hbm, params_hbm, out_hbm,
              item_v, row_v, ig_v, gg_v, params_v, wsplat_v, raw_v, out_v,
              sem):
    wid = lax.axis_index("s") * _NC + lax.axis_index("c")
    base = wid * _BPW
    pltpu.sync_copy(item_hbm.at[pl.ds(base, _BPW)], item_v)
    pltpu.sync_copy(ig_hbm.at[pl.ds(base, _BPW)], ig_v)
    pltpu.sync_copy(gg_hbm.at[pl.ds(base, _BPW)], gg_v)
    pltpu.sync_copy(params_hbm, params_v)

    def _rows(i, _):
        item16 = item_v[pl.ds(i * 16, 16)]
        r128 = (lax.shift_left(lax.shift_right_logical(item16, 13), 10)
                + (item16 & 1023))
        row_v[pl.ds(i * 16, 16)] = (
            lax.shift_left(r128, 3)
            + (lax.shift_right_logical(item16, 10) & 7))
        return ()

    lax.fori_loop(0, _BPW // 16, _rows, (), unroll=4)

    gather = pltpu.async_copy(table_hbm.at[row_v], raw_v, sem)

    def _splat(ck, _):
        wsplat_v[ck, :] = plsc.load_gather(params_v, [jnp.full((16,), ck, jnp.int32)])
        return ()

    lax.fori_loop(0, _EMB * _OUT, _splat, (), unroll=4)

    gather.wait()

    lanes = lax.iota(jnp.int32, 16)

    def _block(b, _):
        rows16 = b * 16 + lanes
        cols = [
            jnp.maximum(
                plsc.load_gather(raw_v, [rows16, jnp.full((16,), k, jnp.int32)]),
                0.0)
            for k in range(_EMB)
        ]
        ig16 = ig_v[pl.ds(b * 16, 16)]
        gg16 = gg_v[pl.ds(b * 16, 16)]
        for c in range(_OUT):
            acc = plsc.load_gather(params_v, [_WIG + c * _NIG + ig16])
            acc = acc + plsc.load_gather(params_v, [_WGG + c * _NGG + gg16])
            for k in range(_EMB):
                acc = acc + cols[k] * wsplat_v[c * _EMB + k, :]
            out_v[c, pl.ds(b * 16, 16)] = acc
        return ()

    lax.fori_loop(0, _NBLK, _block, ())

    pltpu.sync_copy(out_v, out_hbm.at[:, pl.ds(base, _BPW)])


@jax.jit
def kernel(item_indices, index_group_indices, garment_group_indices, emb_table, W, b):
    item = item_indices.astype(jnp.int32)
    ig = index_group_indices.astype(jnp.int32)
    gg = garment_group_indices.astype(jnp.int32)
    t16 = _repack(emb_table.T).reshape(-1, _EMB)
    w0t = W[:_EMB].T.reshape(-1)
    wigb = (W[_EMB:_EMB + _NIG] + b[None, :]).T.reshape(-1)
    wggt = W[_EMB + _NIG:].T.reshape(-1)
    params = jnp.zeros((1024,), jnp.float32)
    params = lax.dynamic_update_slice(params, w0t, (_W0,))
    params = lax.dynamic_update_slice(params, wigb, (_WIG,))
    params = lax.dynamic_update_slice(params, wggt, (_WGG,))
    out_t = _sc_tower(t16, item, ig, gg, params)
    return out_t.T


# speedup vs baseline: 2.2123x; 1.4030x over previous
import functools

import jax
import jax.numpy as jnp
from jax import lax
from jax.experimental import pallas as pl
from jax.experimental.pallas import tpu as pltpu
from jax.experimental.pallas import tpu_sc as plsc

_BATCH = 16384
_EMB = 16
_OUT = 10
_NIG = 10
_NGG = 21
_TBLK = 8192
_TGRID = 13
_TROWS = _TGRID * _TBLK // 8
_SLOT = _TBLK // 8

_info = plsc.get_sparse_core_info()
_NC = _info.num_cores
_NS = _info.num_subcores
_NW = _NC * _NS
_BPW = _BATCH // _NW
_NBLK = _BPW // 16

_W0 = 0
_WIG = 256
_WGG = 512


def _repack_body(t_ref, o_ref):
    x = t_ref[...].reshape(128, _SLOT)
    y = jnp.transpose(x)
    li = lax.broadcasted_iota(jnp.int32, (_SLOT, 128), 1)
    perm = (li % 16) * 8 + li // 16
    o_ref[...] = jnp.take_along_axis(y, perm, axis=1)


def _repack(tT):
    return pl.pallas_call(
        _repack_body,
        grid=(_TGRID,),
        in_specs=[pl.BlockSpec((_EMB, _TBLK), lambda i: (0, i))],
        out_specs=pl.BlockSpec((_TBLK // 8, 128), lambda i: (i, 0)),
        out_shape=jax.ShapeDtypeStruct((_TROWS, 128), jnp.float32),
    )(tT)


_sc_mesh = plsc.VectorSubcoreMesh(core_axis_name="c", subcore_axis_name="s")


@functools.partial(
    pl.kernel,
    mesh=_sc_mesh,
    out_type=jax.ShapeDtypeStruct((_OUT, _BATCH), jnp.float32),
    scratch_types=[
        pltpu.VMEM((_BPW,), jnp.int32),
        pltpu.VMEM((_BPW,), jnp.int32),
        pltpu.VMEM((_BPW,), jnp.int32),
        pltpu.VMEM((_BPW,), jnp.int32),
        pltpu.VMEM((1024,), jnp.float32),
        pltpu.VMEM((_EMB * _OUT, 16), jnp.float32),
        pltpu.VMEM((_BPW, _EMB), jnp.float32),
        pltpu.VMEM((_OUT, _BPW), jnp.float32),
        pltpu.SemaphoreType.DMA,
    ],
    compiler_params=pltpu.CompilerParams(
        use_tc_tiling_on_sc=False, needs_layout_passes=False),
)
def _sc_tower(table_hbm, item_hbm, ig_hbm, gg_hbm, params_hbm, out_hbm,
              item_v, row_v, ig_v, gg_v, params_v, wsplat_v, raw_v, out_v,
              sem):
    wid = lax.axis_index("s") * _NC + lax.axis_index("c")
    base = wid * _BPW
    pltpu.sync_copy(item_hbm.at[pl.ds(base, _BPW)], item_v)
    pltpu.sync_copy(ig_hbm.at[pl.ds(base, _BPW)], ig_v)
    pltpu.sync_copy(gg_hbm.at[pl.ds(base, _BPW)], gg_v)
    pltpu.sync_copy(params_hbm, params_v)

    def _rows(i, _):
        item16 = item_v[pl.ds(i * 16, 16)]
        r128 = (lax.shift_left(lax.shift_right_logical(item16, 13), 10)
                + (item16 & 1023))
        row_v[pl.ds(i * 16, 16)] = (
            lax.shift_left(r128, 3)
            + (lax.shift_right_logical(item16, 10) & 7))
        return ()

    lax.fori_loop(0, _BPW // 16, _rows, (), unroll=4)

    gather = pltpu.async_copy(table_hbm.at[row_v], raw_v, sem)

    def _splat(ck, _):
        wsplat_v[ck, :] = plsc.load_gather(params_v, [jnp.full((16,), ck, jnp.int32)])
        return ()

    lax.fori_loop(0, _EMB * _OUT, _splat, (), unroll=4)

    gather.wait()

    lanes = lax.iota(jnp.int32, 16)

    def _block(b, _):
        rows16 = b * 16 + lanes
        cols = [
            jnp.maximum(
                plsc.load_gather(raw_v, [rows16, jnp.full((16,), k, jnp.int32)]),
                0.0)
            for k in range(_EMB)
        ]
        ig16 = ig_v[pl.ds(b * 16, 16)]
        gg16 = gg_v[pl.ds(b * 16, 16)]
        for c in range(_OUT):
            terms = [cols[k] * wsplat_v[c * _EMB + k, :] for k in range(_EMB)]
            terms.append(plsc.load_gather(params_v, [_WIG + c * _NIG + ig16]))
            terms.append(plsc.load_gather(params_v, [_WGG + c * _NGG + gg16]))
            while len(terms) > 1:
                terms = [terms[i] + terms[i + 1] for i in range(0, len(terms) - 1, 2)] + (
                    [terms[-1]] if len(terms) % 2 else [])
            out_v[c, pl.ds(b * 16, 16)] = terms[0]
        return ()

    lax.fori_loop(0, _NBLK, _block, ())

    pltpu.sync_copy(out_v, out_hbm.at[:, pl.ds(base, _BPW)])


@jax.jit
def kernel(item_indices, index_group_indices, garment_group_indices, emb_table, W, b):
    item = item_indices.astype(jnp.int32)
    ig = index_group_indices.astype(jnp.int32)
    gg = garment_group_indices.astype(jnp.int32)
    t16 = _repack(emb_table.T).reshape(-1, _EMB)
    w0t = W[:_EMB].T.reshape(-1)
    wigb = (W[_EMB:_EMB + _NIG] + b[None, :]).T.reshape(-1)
    wggt = W[_EMB + _NIG:].T.reshape(-1)
    params = jnp.zeros((1024,), jnp.float32)
    params = lax.dynamic_update_slice(params, w0t, (_W0,))
    params = lax.dynamic_update_slice(params, wigb, (_WIG,))
    params = lax.dynamic_update_slice(params, wggt, (_WGG,))
    out_t = _sc_tower(t16, item, ig, gg, params)
    return out_t.T


# speedup vs baseline: 2.3572x; 1.0655x over previous
import functools

import jax
import jax.numpy as jnp
from jax import lax
from jax.experimental import pallas as pl
from jax.experimental.pallas import tpu as pltpu
from jax.experimental.pallas import tpu_sc as plsc

_BATCH = 16384
_EMB = 16
_OUT = 10
_NIG = 10
_NGG = 21
_TBLK = 8192
_TGRID = 13
_TROWS = _TGRID * _TBLK // 8
_SLOT = _TBLK // 8

_info = plsc.get_sparse_core_info()
_NC = _info.num_cores
_NS = _info.num_subcores
_NW = _NC * _NS
_BPW = _BATCH // _NW
_NBLK = _BPW // 16

_W0 = 0
_WIG = 256
_WGG = 512


def _repack_body(t_ref, w_ref, o_ref):
    x = t_ref[...].reshape(128, _SLOT)
    y = jnp.transpose(x)
    li = lax.broadcasted_iota(jnp.int32, (_SLOT, 128), 1)
    perm = (li % 16) * 8 + li // 16
    feats = jnp.take_along_axis(y, perm, axis=1)
    o_ref[...] = jnp.dot(jnp.maximum(feats, 0.0), w_ref[...],
                         preferred_element_type=jnp.float32)


def _repack(tT, wbd):
    return pl.pallas_call(
        _repack_body,
        grid=(_TGRID,),
        in_specs=[pl.BlockSpec((_EMB, _TBLK), lambda i: (0, i)),
                  pl.BlockSpec((128, 128), lambda i: (0, 0))],
        out_specs=pl.BlockSpec((_TBLK // 8, 128), lambda i: (i, 0)),
        out_shape=jax.ShapeDtypeStruct((_TROWS, 128), jnp.float32),
    )(tT, wbd)


_sc_mesh = plsc.VectorSubcoreMesh(core_axis_name="c", subcore_axis_name="s")


@functools.partial(
    pl.kernel,
    mesh=_sc_mesh,
    out_type=jax.ShapeDtypeStruct((_OUT, _BATCH), jnp.float32),
    scratch_types=[
        pltpu.VMEM((_BPW,), jnp.int32),
        pltpu.VMEM((_BPW,), jnp.int32),
        pltpu.VMEM((_BPW,), jnp.int32),
        pltpu.VMEM((_BPW,), jnp.int32),
        pltpu.VMEM((1024,), jnp.float32),
        pltpu.VMEM((_BPW, _EMB), jnp.float32),
        pltpu.VMEM((_OUT, _BPW), jnp.float32),
        pltpu.SemaphoreType.DMA,
    ],
    compiler_params=pltpu.CompilerParams(
        use_tc_tiling_on_sc=False, needs_layout_passes=False),
)
def _sc_tower(table_hbm, item_hbm, ig_hbm, gg_hbm, params_hbm, out_hbm,
              item_v, row_v, ig_v, gg_v, params_v, raw_v, out_v, sem):
    wid = lax.axis_index("s") * _NC + lax.axis_index("c")
    base = wid * _BPW
    pltpu.sync_copy(item_hbm.at[pl.ds(base, _BPW)], item_v)
    pltpu.sync_copy(ig_hbm.at[pl.ds(base, _BPW)], ig_v)
    pltpu.sync_copy(gg_hbm.at[pl.ds(base, _BPW)], gg_v)
    pltpu.sync_copy(params_hbm, params_v)

    def _rows(i, _):
        item16 = item_v[pl.ds(i * 16, 16)]
        r128 = (lax.shift_left(lax.shift_right_logical(item16, 13), 10)
                + (item16 & 1023))
        row_v[pl.ds(i * 16, 16)] = (
            lax.shift_left(r128, 3)
            + (lax.shift_right_logical(item16, 10) & 7))
        return ()

    lax.fori_loop(0, _BPW // 16, _rows, (), unroll=4)

    pltpu.async_copy(table_hbm.at[row_v], raw_v, sem).wait()

    lanes = lax.iota(jnp.int32, 16)

    def _block(b, _):
        rows16 = b * 16 + lanes
        ig16 = ig_v[pl.ds(b * 16, 16)]
        gg16 = gg_v[pl.ds(b * 16, 16)]
        for c in range(_OUT):
            emb_c = plsc.load_gather(raw_v, [rows16, jnp.full((16,), c, jnp.int32)])
            wib_c = plsc.load_gather(params_v, [_WIG + c * _NIG + ig16])
            wgg_c = plsc.load_gather(params_v, [_WGG + c * _NGG + gg16])
            out_v[c, pl.ds(b * 16, 16)] = (emb_c + wib_c) + wgg_c
        return ()

    lax.fori_loop(0, _NBLK, _block, ())

    pltpu.sync_copy(out_v, out_hbm.at[:, pl.ds(base, _BPW)])


@jax.jit
def kernel(item_indices, index_group_indices, garment_group_indices, emb_table, W, b):
    item = item_indices.astype(jnp.int32)
    ig = index_group_indices.astype(jnp.int32)
    gg = garment_group_indices.astype(jnp.int32)
    w0p = jnp.pad(W[:_EMB], ((0, 0), (0, 6)))
    wbd = jnp.kron(jnp.eye(8, dtype=jnp.float32), w0p)
    t16 = _repack(emb_table.T, wbd).reshape(-1, _EMB)
    wigb = (W[_EMB:_EMB + _NIG] + b[None, :]).T.reshape(-1)
    wggt = W[_EMB + _NIG:].T.reshape(-1)
    params = jnp.zeros((1024,), jnp.float32)
    params = lax.dynamic_update_slice(params, wigb, (_WIG,))
    params = lax.dynamic_update_slice(params, wggt, (_WGG,))
    out_t = _sc_tower(t16, item, ig, gg, params)
    return out_t.T


# speedup vs baseline: 2.5360x; 1.0759x over previous
import functools

import jax
import jax.numpy as jnp
from jax import lax
from jax.experimental import pallas as pl
from jax.experimental.pallas import tpu as pltpu
from jax.experimental.pallas import tpu_sc as plsc

_BATCH = 16384
_EMB = 16
_OUT = 10
_NIG = 10
_NGG = 21
_TBLK = 16384
_TGRID = 7
_TROWS = _TGRID * _TBLK // 8
_SLOT = _TBLK // 8

_info = plsc.get_sparse_core_info()
_NC = _info.num_cores
_NS = _info.num_subcores
_NW = _NC * _NS
_BPW = _BATCH // _NW
_NBLK = _BPW // 16

_W0 = 0
_WIG = 256
_WGG = 512


def _repack_body(t_ref, w_ref, o_ref):
    x = t_ref[...].reshape(128, _SLOT)
    y = jnp.transpose(x)
    li = lax.broadcasted_iota(jnp.int32, (_SLOT, 128), 1)
    perm = (li % 16) * 8 + li // 16
    feats = jnp.take_along_axis(y, perm, axis=1)
    o_ref[...] = jnp.dot(jnp.maximum(feats, 0.0), w_ref[...],
                         preferred_element_type=jnp.float32)


def _repack(tT, wbd):
    return pl.pallas_call(
        _repack_body,
        grid=(_TGRID,),
        in_specs=[pl.BlockSpec((_EMB, _TBLK), lambda i: (0, i)),
                  pl.BlockSpec((128, 128), lambda i: (0, 0))],
        out_specs=pl.BlockSpec((_TBLK // 8, 128), lambda i: (i, 0)),
        out_shape=jax.ShapeDtypeStruct((_TROWS, 128), jnp.float32),
    )(tT, wbd)


_sc_mesh = plsc.VectorSubcoreMesh(core_axis_name="c", subcore_axis_name="s")


@functools.partial(
    pl.kernel,
    mesh=_sc_mesh,
    out_type=jax.ShapeDtypeStruct((_OUT, _BATCH), jnp.float32),
    scratch_types=[
        pltpu.VMEM((_BPW,), jnp.int32),
        pltpu.VMEM((_BPW,), jnp.int32),
        pltpu.VMEM((_BPW,), jnp.int32),
        pltpu.VMEM((_BPW,), jnp.int32),
        pltpu.VMEM((1024,), jnp.float32),
        pltpu.VMEM((_BPW, _EMB), jnp.float32),
        pltpu.VMEM((_OUT, _BPW), jnp.float32),
        pltpu.SemaphoreType.DMA,
    ],
    compiler_params=pltpu.CompilerParams(
        use_tc_tiling_on_sc=False, needs_layout_passes=False),
)
def _sc_tower(table_hbm, item_hbm, ig_hbm, gg_hbm, params_hbm, out_hbm,
              item_v, row_v, ig_v, gg_v, params_v, raw_v, out_v, sem):
    wid = lax.axis_index("s") * _NC + lax.axis_index("c")
    base = wid * _BPW
    pltpu.sync_copy(item_hbm.at[pl.ds(base, _BPW)], item_v)
    pltpu.sync_copy(ig_hbm.at[pl.ds(base, _BPW)], ig_v)
    pltpu.sync_copy(gg_hbm.at[pl.ds(base, _BPW)], gg_v)
    pltpu.sync_copy(params_hbm, params_v)

    def _rows(i, _):
        item16 = item_v[pl.ds(i * 16, 16)]
        r128 = (lax.shift_left(lax.shift_right_logical(item16, 14), 11)
                + (item16 & 2047))
        row_v[pl.ds(i * 16, 16)] = (
            lax.shift_left(r128, 3)
            + (lax.shift_right_logical(item16, 11) & 7))
        return ()

    lax.fori_loop(0, _BPW // 16, _rows, (), unroll=4)

    pltpu.async_copy(table_hbm.at[row_v], raw_v, sem).wait()

    lanes = lax.iota(jnp.int32, 16)

    def _block(b, _):
        rows16 = b * 16 + lanes
        ig16 = ig_v[pl.ds(b * 16, 16)]
        gg16 = gg_v[pl.ds(b * 16, 16)]
        for c in range(_OUT):
            emb_c = plsc.load_gather(raw_v, [rows16, jnp.full((16,), c, jnp.int32)])
            wib_c = plsc.load_gather(params_v, [_WIG + c * _NIG + ig16])
            wgg_c = plsc.load_gather(params_v, [_WGG + c * _NGG + gg16])
            out_v[c, pl.ds(b * 16, 16)] = (emb_c + wib_c) + wgg_c
        return ()

    lax.fori_loop(0, _NBLK, _block, (), unroll=2)

    pltpu.sync_copy(out_v, out_hbm.at[:, pl.ds(base, _BPW)])


@jax.jit
def kernel(item_indices, index_group_indices, garment_group_indices, emb_table, W, b):
    item = item_indices.astype(jnp.int32)
    ig = index_group_indices.astype(jnp.int32)
    gg = garment_group_indices.astype(jnp.int32)
    w0p = jnp.pad(W[:_EMB], ((0, 0), (0, 6)))
    wbd = jnp.kron(jnp.eye(8, dtype=jnp.float32), w0p)
    t16 = _repack(emb_table.T, wbd).reshape(-1, _EMB)
    wigb = (W[_EMB:_EMB + _NIG] + b[None, :]).T.reshape(-1)
    wggt = W[_EMB + _NIG:].T.reshape(-1)
    params = jnp.zeros((1024,), jnp.float32)
    params = lax.dynamic_update_slice(params, wigb, (_WIG,))
    params = lax.dynamic_update_slice(params, wggt, (_WGG,))
    out_t = _sc_tower(t16, item, ig, gg, params)
    return out_t.T


# speedup vs baseline: 2.5478x; 1.0046x over previous
import functools

import jax
import jax.numpy as jnp
from jax import lax
from jax.experimental import pallas as pl
from jax.experimental.pallas import tpu as pltpu
from jax.experimental.pallas import tpu_sc as plsc

_BATCH = 16384
_EMB = 16
_OUT = 10
_NIG = 10
_NGG = 21
_TBLK = 32768
_TGRID = 4
_TROWS = _TGRID * _TBLK // 8
_SLOT = _TBLK // 8

_info = plsc.get_sparse_core_info()
_NC = _info.num_cores
_NS = _info.num_subcores
_NW = _NC * _NS
_BPW = _BATCH // _NW
_NBLK = _BPW // 16

_W0 = 0
_WIG = 256
_WGG = 512


def _repack_body(t_ref, w_ref, o_ref):
    x = t_ref[...].reshape(128, _SLOT)
    y = jnp.transpose(x)
    li = lax.broadcasted_iota(jnp.int32, (_SLOT, 128), 1)
    perm = (li % 16) * 8 + li // 16
    feats = jnp.take_along_axis(y, perm, axis=1)
    o_ref[...] = jnp.dot(jnp.maximum(feats, 0.0), w_ref[...],
                         preferred_element_type=jnp.float32)


def _repack(tT, wbd):
    return pl.pallas_call(
        _repack_body,
        grid=(_TGRID,),
        in_specs=[pl.BlockSpec((_EMB, _TBLK), lambda i: (0, i)),
                  pl.BlockSpec((128, 128), lambda i: (0, 0))],
        out_specs=pl.BlockSpec((_TBLK // 8, 128), lambda i: (i, 0)),
        out_shape=jax.ShapeDtypeStruct((_TROWS, 128), jnp.float32),
    )(tT, wbd)


_sc_mesh = plsc.VectorSubcoreMesh(core_axis_name="c", subcore_axis_name="s")


@functools.partial(
    pl.kernel,
    mesh=_sc_mesh,
    out_type=jax.ShapeDtypeStruct((_OUT, _BATCH), jnp.float32),
    scratch_types=[
        pltpu.VMEM((_BPW,), jnp.int32),
        pltpu.VMEM((_BPW,), jnp.int32),
        pltpu.VMEM((_BPW,), jnp.int32),
        pltpu.VMEM((_BPW,), jnp.int32),
        pltpu.VMEM((1024,), jnp.float32),
        pltpu.VMEM((_BPW, _EMB), jnp.float32),
        pltpu.VMEM((_OUT, _BPW), jnp.float32),
        pltpu.SemaphoreType.DMA,
    ],
    compiler_params=pltpu.CompilerParams(
        use_tc_tiling_on_sc=False, needs_layout_passes=False),
)
def _sc_tower(table_hbm, item_hbm, ig_hbm, gg_hbm, params_hbm, out_hbm,
              item_v, row_v, ig_v, gg_v, params_v, raw_v, out_v, sem):
    wid = lax.axis_index("s") * _NC + lax.axis_index("c")
    base = wid * _BPW
    pltpu.sync_copy(item_hbm.at[pl.ds(base, _BPW)], item_v)
    pltpu.sync_copy(ig_hbm.at[pl.ds(base, _BPW)], ig_v)
    pltpu.sync_copy(gg_hbm.at[pl.ds(base, _BPW)], gg_v)
    pltpu.sync_copy(params_hbm, params_v)

    def _rows(i, _):
        item16 = item_v[pl.ds(i * 16, 16)]
        r128 = (lax.shift_left(lax.shift_right_logical(item16, 15), 12)
                + (item16 & 4095))
        row_v[pl.ds(i * 16, 16)] = (
            lax.shift_left(r128, 3)
            + (lax.shift_right_logical(item16, 12) & 7))
        return ()

    lax.fori_loop(0, _BPW // 16, _rows, (), unroll=4)

    pltpu.async_copy(table_hbm.at[row_v], raw_v, sem).wait()

    lanes = lax.iota(jnp.int32, 16)

    def _block(b, _):
        rows16 = b * 16 + lanes
        ig16 = ig_v[pl.ds(b * 16, 16)]
        gg16 = gg_v[pl.ds(b * 16, 16)]
        for c in range(_OUT):
            emb_c = plsc.load_gather(raw_v, [rows16, jnp.full((16,), c, jnp.int32)])
            wib_c = plsc.load_gather(params_v, [_WIG + c * _NIG + ig16])
            wgg_c = plsc.load_gather(params_v, [_WGG + c * _NGG + gg16])
            out_v[c, pl.ds(b * 16, 16)] = (emb_c + wib_c) + wgg_c
        return ()

    lax.fori_loop(0, _NBLK, _block, (), unroll=2)

    pltpu.sync_copy(out_v, out_hbm.at[:, pl.ds(base, _BPW)])


@jax.jit
def kernel(item_indices, index_group_indices, garment_group_indices, emb_table, W, b):
    item = item_indices.astype(jnp.int32)
    ig = index_group_indices.astype(jnp.int32)
    gg = garment_group_indices.astype(jnp.int32)
    w0p = jnp.pad(W[:_EMB], ((0, 0), (0, 6)))
    wbd = jnp.kron(jnp.eye(8, dtype=jnp.float32), w0p)
    t16 = _repack(emb_table.T, wbd).reshape(-1, _EMB)
    wigb = (W[_EMB:_EMB + _NIG] + b[None, :]).T.reshape(-1)
    wggt = W[_EMB + _NIG:].T.reshape(-1)
    params = jnp.zeros((1024,), jnp.float32)
    params = lax.dynamic_update_slice(params, wigb, (_WIG,))
    params = lax.dynamic_update_slice(params, wggt, (_WGG,))
    out_t = _sc_tower(t16, item, ig, gg, params)
    return out_t.T
